# baseline (speedup 1.0000x reference)
import jax
import jax.numpy as jnp
from jax.experimental import pallas as pl
from jax.experimental.pallas import tpu as pltpu

_B, _S, _D = 4, 8192, 1024
_NUM_TYPES = 8
_BLOCK = 1024


def _add_kernel(idx_ref, table_ref, x_ref, o_ref):
    row = table_ref[pl.ds(idx_ref[0], 1), :]
    o_ref[...] = x_ref[...] + row


def kernel(x, type_id, type_embedding):
    xf = x.reshape(_B * _S, _D)
    idx = jnp.asarray(type_id, jnp.int32).reshape(1)
    grid = (_B * _S // _BLOCK,)
    out = pl.pallas_call(
        _add_kernel,
        grid_spec=pltpu.PrefetchScalarGridSpec(
            num_scalar_prefetch=1,
            grid=grid,
            in_specs=[
                pl.BlockSpec((_NUM_TYPES, _D), lambda i, idx: (0, 0)),
                pl.BlockSpec((_BLOCK, _D), lambda i, idx: (i, 0)),
            ],
            out_specs=pl.BlockSpec((_BLOCK, _D), lambda i, idx: (i, 0)),
        ),
        out_shape=jax.ShapeDtypeStruct((_B * _S, _D), jnp.float32),
        compiler_params=pltpu.CompilerParams(
            dimension_semantics=("parallel",),
        ),
    )(idx, type_embedding, xf)
    return out.reshape(_B, _S, _D)


# speedup vs baseline: 1.0224x; 1.0224x over previous
import jax
import jax.numpy as jnp
from jax.experimental import pallas as pl
from jax.experimental.pallas import tpu as pltpu

_B, _S, _D = 4, 8192, 1024
_NUM_TYPES = 8
_BLOCK = 2048


def _add_kernel(idx_ref, table_ref, x_ref, o_ref):
    row = table_ref[pl.ds(idx_ref[0], 1), :]
    o_ref[...] = x_ref[...] + row


def kernel(x, type_id, type_embedding):
    xf = x.reshape(_B * _S, _D)
    idx = jnp.asarray(type_id, jnp.int32).reshape(1)
    grid = (_B * _S // _BLOCK,)
    out = pl.pallas_call(
        _add_kernel,
        grid_spec=pltpu.PrefetchScalarGridSpec(
            num_scalar_prefetch=1,
            grid=grid,
            in_specs=[
                pl.BlockSpec((_NUM_TYPES, _D), lambda i, idx: (0, 0)),
                pl.BlockSpec((_BLOCK, _D), lambda i, idx: (i, 0)),
            ],
            out_specs=pl.BlockSpec((_BLOCK, _D), lambda i, idx: (i, 0)),
        ),
        out_shape=jax.ShapeDtypeStruct((_B * _S, _D), jnp.float32),
        compiler_params=pltpu.CompilerParams(
            dimension_semantics=("parallel",),
        ),
    )(idx, type_embedding, xf)
    return out.reshape(_B, _S, _D)
